# Initial kernel scaffold; baseline (speedup 1.0000x reference)
import jax
import jax.numpy as jnp
from jax.experimental import pallas as pl


def kernel(state, edge_index, W1_rel, b1_rel, W1_root, W2_rel, b2_rel, W2_root):
    raise NotImplementedError("write your pallas kernel here")


# speedup vs baseline: 4.9163x; 4.9163x over previous
import functools

import jax
import jax.numpy as jnp
from jax import lax
from jax.experimental import pallas as pl
from jax.experimental.pallas import tpu as pltpu
from jax.experimental.pallas import tpu_sc as plsc

N = 50000
E = 1600000
D = 33
DA = 32
DB = 8

NC = 2
NS = 16
NW = NC * NS
EW = E // NW
K = 80
NCHUNK = EW // K
ZR = 3128


def _make_sc_scatter():
    mesh = plsc.VectorSubcoreMesh(core_axis_name="c", subcore_axis_name="s")

    @functools.partial(
        pl.kernel,
        out_type=[
            jax.ShapeDtypeStruct((NC, N, DA), jnp.float32),
            jax.ShapeDtypeStruct((NC, N, DB), jnp.float32),
        ],
        mesh=mesh,
        compiler_params=pltpu.CompilerParams(use_tc_tiling_on_sc=False),
        scratch_types=[
            pltpu.VMEM((K,), jnp.int32),
            pltpu.VMEM((K,), jnp.int32),
            pltpu.VMEM((K, DA), jnp.float32),
            pltpu.VMEM((K, DB), jnp.float32),
            pltpu.VMEM_SHARED((N, DA), jnp.float32),
            pltpu.VMEM_SHARED((N, DB), jnp.float32),
            pltpu.SemaphoreType.DMA,
        ],
    )
    def sc_scatter(state_a_hbm, state_b_hbm, src_hbm, dst_hbm,
                   zeros_a_hbm, zeros_b_hbm, out_a_hbm, out_b_hbm,
                   idx_s, idx_d, rows_a, rows_b, acc_a, acc_b, sem):
        c = lax.axis_index("c")
        s = lax.axis_index("s")
        wid = s * NC + c

        r0 = jnp.minimum(s * ZR, N - ZR)
        pltpu.sync_copy(zeros_a_hbm.at[pl.ds(r0, ZR)], acc_a.at[pl.ds(r0, ZR)])
        pltpu.sync_copy(zeros_b_hbm.at[pl.ds(r0, ZR)], acc_b.at[pl.ds(r0, ZR)])
        plsc.subcore_barrier()

        base = wid * EW

        def body(i, carry):
            off = base + i * K
            pltpu.sync_copy(src_hbm.at[pl.ds(off, K)], idx_s)
            pltpu.sync_copy(dst_hbm.at[pl.ds(off, K)], idx_d)
            pltpu.async_copy(state_a_hbm.at[idx_s], rows_a, sem).wait()
            pltpu.async_copy(state_b_hbm.at[idx_s], rows_b, sem).wait()
            pltpu.sync_copy(rows_a, acc_a.at[idx_d], add=True)
            pltpu.sync_copy(rows_b, acc_b.at[idx_d], add=True)
            return carry

        lax.fori_loop(0, NCHUNK, body, 0)
        plsc.subcore_barrier()

        pltpu.sync_copy(acc_a.at[pl.ds(r0, ZR)], out_a_hbm.at[c, pl.ds(r0, ZR)])
        pltpu.sync_copy(acc_b.at[pl.ds(r0, ZR)], out_b_hbm.at[c, pl.ds(r0, ZR)])

    return sc_scatter


_sc_scatter = _make_sc_scatter()


def _tc_combine_body(acc_a_ref, acc_b_ref, state_ref, wa_ref, wb_ref,
                     wroot_ref, b_ref, out_ref):
    agg_a = acc_a_ref[0] + acc_a_ref[1]
    agg_b = acc_b_ref[0] + acc_b_ref[1]
    out = jnp.dot(agg_a, wa_ref[...], preferred_element_type=jnp.float32)
    out += jnp.dot(agg_b, wb_ref[...], preferred_element_type=jnp.float32)
    out += jnp.dot(state_ref[...], wroot_ref[...],
                   preferred_element_type=jnp.float32)
    out += b_ref[...]
    out_ref[...] = jnp.maximum(out, 0.0)


BN = 5000


def _tc_combine(pa, pb, state, w_a, w_b, w_root, b):
    grid = (N // BN,)
    return pl.pallas_call(
        _tc_combine_body,
        grid=grid,
        in_specs=[
            pl.BlockSpec((NC, BN, DA), lambda i: (0, i, 0)),
            pl.BlockSpec((NC, BN, DB), lambda i: (0, i, 0)),
            pl.BlockSpec((BN, D), lambda i: (i, 0)),
            pl.BlockSpec((DA, D), lambda i: (0, 0)),
            pl.BlockSpec((DB, D), lambda i: (0, 0)),
            pl.BlockSpec((D, D), lambda i: (0, 0)),
            pl.BlockSpec((1, D), lambda i: (0, 0)),
        ],
        out_specs=pl.BlockSpec((BN, D), lambda i: (i, 0)),
        out_shape=jax.ShapeDtypeStruct((N, D), jnp.float32),
    )(pa, pb, state, w_a, w_b, w_root, b)


def kernel(state, edge_index, W1_rel, b1_rel, W1_root, W2_rel, b2_rel, W2_root):
    del W1_rel, b1_rel, W1_root
    src = edge_index[0].astype(jnp.int32)
    dst = edge_index[1].astype(jnp.int32)
    state_a = state[:, :DA]
    state_b = jnp.pad(state[:, DA:], ((0, 0), (0, DB - (D - DA))))
    zeros_a = jnp.zeros((N, DA), jnp.float32)
    zeros_b = jnp.zeros((N, DB), jnp.float32)
    pa, pb = _sc_scatter(state_a, state_b, src, dst, zeros_a, zeros_b)
    w_a = W2_rel[:DA]
    w_b = jnp.pad(W2_rel[DA:], ((0, DB - (D - DA)), (0, 0)))
    return _tc_combine(pa, pb, state, w_a, w_b, W2_root,
                       b2_rel.reshape(1, D))


# speedup vs baseline: 6.6160x; 1.3457x over previous
import functools

import jax
import jax.numpy as jnp
from jax import lax
from jax.experimental import pallas as pl
from jax.experimental.pallas import tpu as pltpu
from jax.experimental.pallas import tpu_sc as plsc

N = 50000
E = 1600000
D = 33
DP = 40

NC = 2
NS = 16
NW = NC * NS
EW = E // NW
K = 80
NCHUNK = EW // K
ZR = 3128


def _make_sc_scatter():
    mesh = plsc.VectorSubcoreMesh(core_axis_name="c", subcore_axis_name="s")

    @functools.partial(
        pl.kernel,
        out_type=jax.ShapeDtypeStruct((NC, N, DP), jnp.float32),
        mesh=mesh,
        compiler_params=pltpu.CompilerParams(use_tc_tiling_on_sc=False),
        scratch_types=[
            pltpu.VMEM((K,), jnp.int32),
            pltpu.VMEM((K,), jnp.int32),
            pltpu.VMEM((K, DP), jnp.float32),
            pltpu.VMEM_SHARED((N, DP), jnp.float32),
            pltpu.SemaphoreType.DMA,
        ],
    )
    def sc_scatter(state_hbm, src_hbm, dst_hbm, zeros_hbm, out_hbm,
                   idx_s, idx_d, rows, acc, sem):
        c = lax.axis_index("c")
        s = lax.axis_index("s")
        wid = s * NC + c

        r0 = jnp.minimum(s * ZR, N - ZR)
        pltpu.sync_copy(zeros_hbm.at[pl.ds(r0, ZR)], acc.at[pl.ds(r0, ZR)])
        plsc.subcore_barrier()

        base = wid * EW

        def body(i, carry):
            off = base + i * K
            pltpu.sync_copy(src_hbm.at[pl.ds(off, K)], idx_s)
            pltpu.sync_copy(dst_hbm.at[pl.ds(off, K)], idx_d)
            pltpu.async_copy(state_hbm.at[idx_s], rows, sem).wait()
            pltpu.sync_copy(rows, acc.at[idx_d], add=True)
            return carry

        lax.fori_loop(0, NCHUNK, body, 0)
        plsc.subcore_barrier()

        pltpu.sync_copy(acc.at[pl.ds(r0, ZR)], out_hbm.at[c, pl.ds(r0, ZR)])

    return sc_scatter


_sc_scatter = _make_sc_scatter()


def _tc_combine_body(acc_ref, state_ref, wrel_ref, wroot_ref, b_ref, out_ref):
    agg = acc_ref[0] + acc_ref[1]
    out = jnp.dot(agg, wrel_ref[...], preferred_element_type=jnp.float32)
    out += jnp.dot(state_ref[...], wroot_ref[...],
                   preferred_element_type=jnp.float32)
    out += b_ref[...]
    out_ref[...] = jnp.maximum(out, 0.0)


BN = 5000


def _tc_combine(pacc, state, w_rel, w_root, b):
    grid = (N // BN,)
    return pl.pallas_call(
        _tc_combine_body,
        grid=grid,
        in_specs=[
            pl.BlockSpec((NC, BN, DP), lambda i: (0, i, 0)),
            pl.BlockSpec((BN, D), lambda i: (i, 0)),
            pl.BlockSpec((DP, D), lambda i: (0, 0)),
            pl.BlockSpec((D, D), lambda i: (0, 0)),
            pl.BlockSpec((1, D), lambda i: (0, 0)),
        ],
        out_specs=pl.BlockSpec((BN, D), lambda i: (i, 0)),
        out_shape=jax.ShapeDtypeStruct((N, D), jnp.float32),
    )(pacc, state, w_rel, w_root, b)


def kernel(state, edge_index, W1_rel, b1_rel, W1_root, W2_rel, b2_rel, W2_root):
    del W1_rel, b1_rel, W1_root
    src = edge_index[0].astype(jnp.int32)
    dst = edge_index[1].astype(jnp.int32)
    state_p = jnp.pad(state, ((0, 0), (0, DP - D)))
    zeros_p = jnp.zeros((N, DP), jnp.float32)
    pacc = _sc_scatter(state_p, src, dst, zeros_p)
    w_rel = jnp.pad(W2_rel, ((0, DP - D), (0, 0)))
    return _tc_combine(pacc, state, w_rel, W2_root, b2_rel.reshape(1, D))


# speedup vs baseline: 11.6589x; 1.7622x over previous
import functools

import jax
import jax.numpy as jnp
from jax import lax
from jax.experimental import pallas as pl
from jax.experimental.pallas import tpu as pltpu
from jax.experimental.pallas import tpu_sc as plsc

N = 50000
E = 1600000
D = 33
DP = 40

NC = 2
NS = 16
NW = NC * NS
EW = E // NW
K = 40
NCHUNK = EW // K
SB = 10
SBK = SB * K
NSB = NCHUNK // SB
NB = 3
ZR = 3128


def _make_sc_scatter():
    mesh = plsc.VectorSubcoreMesh(core_axis_name="c", subcore_axis_name="s")

    @functools.partial(
        pl.kernel,
        out_type=jax.ShapeDtypeStruct((NC, N, DP), jnp.float32),
        mesh=mesh,
        compiler_params=pltpu.CompilerParams(use_tc_tiling_on_sc=False),
        scratch_types=[
            pltpu.VMEM((SBK,), jnp.int32),
            pltpu.VMEM((SB, K), jnp.int32),
            pltpu.VMEM((NB, K, DP), jnp.float32),
            pltpu.VMEM_SHARED((N, DP), jnp.float32),
        ] + [pltpu.SemaphoreType.DMA] * NB,
    )
    def sc_scatter(state_hbm, src_hbm, dst2d_hbm, zeros_hbm, out_hbm,
                   idx_s, idx_d, rows, acc, *sems):
        c = lax.axis_index("c")
        s = lax.axis_index("s")
        wid = s * NC + c

        r0 = jnp.minimum(s * ZR, N - ZR)
        pltpu.sync_copy(zeros_hbm.at[pl.ds(r0, ZR)], acc.at[pl.ds(r0, ZR)])
        plsc.subcore_barrier()

        base = wid * EW
        cbase = wid * NCHUNK

        def start_gather(j):
            return pltpu.async_copy(
                state_hbm.at[idx_s.at[pl.ds(j * K, K)]],
                rows.at[j % NB], sems[j % NB])

        def body(sb, carry):
            pltpu.sync_copy(src_hbm.at[pl.ds(base + sb * SBK, SBK)], idx_s)
            pltpu.sync_copy(dst2d_hbm.at[pl.ds(cbase + sb * SB, SB)], idx_d)
            handles = [start_gather(j) for j in range(NB)] + [None] * (SB - NB)
            for j in range(SB):
                handles[j].wait()
                pltpu.sync_copy(rows.at[j % NB], acc.at[idx_d.at[j]], add=True)
                if j + NB < SB:
                    handles[j + NB] = start_gather(j + NB)
            return carry

        lax.fori_loop(0, NSB, body, 0)
        plsc.subcore_barrier()

        pltpu.sync_copy(acc.at[pl.ds(r0, ZR)], out_hbm.at[c, pl.ds(r0, ZR)])

    return sc_scatter


_sc_scatter = _make_sc_scatter()


def _tc_combine_body(acc_ref, state_ref, wrel_ref, wroot_ref, b_ref, out_ref):
    agg = acc_ref[0] + acc_ref[1]
    out = jnp.dot(agg, wrel_ref[...], preferred_element_type=jnp.float32)
    out += jnp.dot(state_ref[...], wroot_ref[...],
                   preferred_element_type=jnp.float32)
    out += b_ref[...]
    out_ref[...] = jnp.maximum(out, 0.0)


BN = 5000


def _tc_combine(pacc, state, w_rel, w_root, b):
    grid = (N // BN,)
    return pl.pallas_call(
        _tc_combine_body,
        grid=grid,
        in_specs=[
            pl.BlockSpec((NC, BN, DP), lambda i: (0, i, 0)),
            pl.BlockSpec((BN, D), lambda i: (i, 0)),
            pl.BlockSpec((DP, D), lambda i: (0, 0)),
            pl.BlockSpec((D, D), lambda i: (0, 0)),
            pl.BlockSpec((1, D), lambda i: (0, 0)),
        ],
        out_specs=pl.BlockSpec((BN, D), lambda i: (i, 0)),
        out_shape=jax.ShapeDtypeStruct((N, D), jnp.float32),
    )(pacc, state, w_rel, w_root, b)


def kernel(state, edge_index, W1_rel, b1_rel, W1_root, W2_rel, b2_rel, W2_root):
    del W1_rel, b1_rel, W1_root
    src = edge_index[0].astype(jnp.int32)
    dst = edge_index[1].astype(jnp.int32).reshape(E // K, K)
    state_p = jnp.pad(state, ((0, 0), (0, DP - D)))
    zeros_p = jnp.zeros((N, DP), jnp.float32)
    pacc = _sc_scatter(state_p, src, dst, zeros_p)
    w_rel = jnp.pad(W2_rel, ((0, DP - D), (0, 0)))
    return _tc_combine(pacc, state, w_rel, W2_root, b2_rel.reshape(1, D))


# speedup vs baseline: 12.6517x; 1.0852x over previous
import functools

import jax
import jax.numpy as jnp
from jax import lax
from jax.experimental import pallas as pl
from jax.experimental.pallas import tpu as pltpu
from jax.experimental.pallas import tpu_sc as plsc

N = 50000
E = 1600000
D = 33
DP = 40

NC = 2
NS = 16
NW = NC * NS
EW = E // NW
K = 40
NCHUNK = EW // K
SB = 10
SBK = SB * K
NSB = NCHUNK // SB
NB = 3
LG = 2
ZR = 3128


def _make_sc_scatter():
    mesh = plsc.VectorSubcoreMesh(core_axis_name="c", subcore_axis_name="s")

    @functools.partial(
        pl.kernel,
        out_type=jax.ShapeDtypeStruct((NC, N, DP), jnp.float32),
        mesh=mesh,
        compiler_params=pltpu.CompilerParams(use_tc_tiling_on_sc=False),
        scratch_types=[
            pltpu.VMEM((SBK,), jnp.int32),
            pltpu.VMEM((SB, K), jnp.int32),
            pltpu.VMEM((NB, K, DP), jnp.float32),
            pltpu.VMEM_SHARED((N, DP), jnp.float32),
            pltpu.SemaphoreType.DMA,
        ] + [pltpu.SemaphoreType.DMA] * (2 * NB),
    )
    def sc_scatter(state_hbm, src_hbm, dst2d_hbm, zeros_hbm, out_hbm,
                   idx_s, idx_d, rows, acc, semi, *sems):
        semg = sems[:NB]
        sems_ = sems[NB:]
        c = lax.axis_index("c")
        s = lax.axis_index("s")
        wid = s * NC + c

        r0 = jnp.minimum(s * ZR, N - ZR)
        pltpu.sync_copy(zeros_hbm, acc.at[pl.ds(r0, ZR)])
        plsc.subcore_barrier()

        base = wid * EW
        cbase = wid * NCHUNK

        def gather(sb, j):
            return pltpu.async_copy(
                state_hbm.at[idx_s.at[pl.ds(j * K, K)]],
                rows.at[j % NB], semg[j % NB])

        def scatter(j):
            return pltpu.async_copy(
                rows.at[j % NB], acc.at[idx_d.at[j]], sems_[j % NB], add=True)

        def body(sb, carry):
            hi1 = pltpu.async_copy(
                src_hbm.at[pl.ds(base + sb * SBK, SBK)], idx_s, semi)
            hi2 = pltpu.async_copy(
                dst2d_hbm.at[pl.ds(cbase + sb * SB, SB)], idx_d, semi)
            hi1.wait()
            hi2.wait()
            hg = [None] * SB
            hs = [None] * SB
            for j in range(SB):
                if j >= NB:
                    hs[j - NB].wait()
                hg[j] = gather(sb, j)
                if j >= LG:
                    hg[j - LG].wait()
                    hs[j - LG] = scatter(j - LG)
            for j in range(SB - LG, SB):
                hg[j].wait()
                hs[j] = scatter(j)
            for j in range(SB - NB, SB):
                hs[j].wait()
            return carry

        lax.fori_loop(0, NSB, body, 0)
        plsc.subcore_barrier()

        pltpu.sync_copy(acc.at[pl.ds(r0, ZR)], out_hbm.at[c, pl.ds(r0, ZR)])

    return sc_scatter


_sc_scatter = _make_sc_scatter()


def _tc_combine_body(acc_ref, state_ref, wrel_ref, wroot_ref, b_ref, out_ref):
    agg = acc_ref[0] + acc_ref[1]
    out = jnp.dot(agg, wrel_ref[...], preferred_element_type=jnp.float32)
    out += jnp.dot(state_ref[...], wroot_ref[...],
                   preferred_element_type=jnp.float32)
    out += b_ref[...]
    out_ref[...] = jnp.maximum(out, 0.0)


BN = 5000


def _tc_combine(pacc, state, w_rel, w_root, b):
    grid = (N // BN,)
    return pl.pallas_call(
        _tc_combine_body,
        grid=grid,
        in_specs=[
            pl.BlockSpec((NC, BN, DP), lambda i: (0, i, 0)),
            pl.BlockSpec((BN, D), lambda i: (i, 0)),
            pl.BlockSpec((DP, D), lambda i: (0, 0)),
            pl.BlockSpec((D, D), lambda i: (0, 0)),
            pl.BlockSpec((1, D), lambda i: (0, 0)),
        ],
        out_specs=pl.BlockSpec((BN, D), lambda i: (i, 0)),
        out_shape=jax.ShapeDtypeStruct((N, D), jnp.float32),
    )(pacc, state, w_rel, w_root, b)


def kernel(state, edge_index, W1_rel, b1_rel, W1_root, W2_rel, b2_rel, W2_root):
    del W1_rel, b1_rel, W1_root
    src = edge_index[0].astype(jnp.int32)
    dst = edge_index[1].astype(jnp.int32).reshape(E // K, K)
    state_p = jnp.pad(state, ((0, 0), (0, DP - D)))
    zeros = jnp.zeros((ZR, DP), jnp.float32)
    pacc = _sc_scatter(state_p, src, dst, zeros)
    w_rel = jnp.pad(W2_rel, ((0, DP - D), (0, 0)))
    return _tc_combine(pacc, state, w_rel, W2_root, b2_rel.reshape(1, D))
